# Initial kernel scaffold; baseline (speedup 1.0000x reference)
import jax
import jax.numpy as jnp
from jax.experimental import pallas as pl


def kernel(features, domains, cluster_probabilities, est_global, est_domains):
    raise NotImplementedError("write your pallas kernel here")


# speedup vs baseline: 46.8570x; 46.8570x over previous
import jax
import jax.numpy as jnp
from jax import lax
from jax.experimental import pallas as pl
from jax.experimental.pallas import tpu as pltpu

_B = 16384
_F = 128
_K = 32
_D = 4
_ALPHA = 0.9
_EPS = 0.001
_TILE = 2048
_NB = _B // _TILE


def _body(f_ref, d_ref, p_ref, eg_ref, ed_ref, out_g_ref, out_d_ref,
          acc_ref, den_ref):
    i = pl.program_id(0)

    @pl.when(i == 0)
    def _init():
        acc_ref[...] = jnp.zeros_like(acc_ref)
        den_ref[...] = jnp.zeros_like(den_ref)

    f = f_ref[...]
    p = p_ref[...]
    dcol = d_ref[...]
    lane_dom = lax.broadcasted_iota(jnp.int32, (1, _D * _K), 1) // _K
    mask = (dcol == lane_dom).astype(jnp.float32)
    p4 = jnp.concatenate([p, p, p, p], axis=1)
    masked = p4 * mask
    acc_ref[...] += lax.dot_general(
        f, masked, (((0,), (0,)), ((), ())),
        preferred_element_type=jnp.float32)
    den_ref[...] += jnp.sum(masked, axis=0, keepdims=True)

    @pl.when(i == _NB - 1)
    def _finish():
        acc = acc_ref[...]
        den = den_ref[...]
        num_g = jnp.zeros((_F, _K), jnp.float32)
        den_g = jnp.zeros((1, _K), jnp.float32)
        for d in range(_D):
            num_d = acc[:, d * _K:(d + 1) * _K]
            den_d = den[:, d * _K:(d + 1) * _K]
            num_g += num_d
            den_g += den_d
            cent_d = num_d / (den_d + _EPS)
            out_d_ref[d] = ed_ref[d] * _ALPHA + cent_d * (1.0 - _ALPHA)
        cent_g = num_g / (den_g + _EPS)
        out_g_ref[...] = eg_ref[...] * _ALPHA + cent_g * (1.0 - _ALPHA)


def kernel(features, domains, cluster_probabilities, est_global, est_domains):
    dcol = domains.reshape(_B, 1)
    out_g, out_d = pl.pallas_call(
        _body,
        grid=(_NB,),
        in_specs=[
            pl.BlockSpec((_TILE, _F), lambda i: (i, 0)),
            pl.BlockSpec((_TILE, 1), lambda i: (i, 0)),
            pl.BlockSpec((_TILE, _K), lambda i: (i, 0)),
            pl.BlockSpec((_F, _K), lambda i: (0, 0)),
            pl.BlockSpec((_D, _F, _K), lambda i: (0, 0, 0)),
        ],
        out_specs=[
            pl.BlockSpec((_F, _K), lambda i: (0, 0)),
            pl.BlockSpec((_D, _F, _K), lambda i: (0, 0, 0)),
        ],
        out_shape=[
            jax.ShapeDtypeStruct((_F, _K), jnp.float32),
            jax.ShapeDtypeStruct((_D, _F, _K), jnp.float32),
        ],
        scratch_shapes=[
            pltpu.VMEM((_F, _D * _K), jnp.float32),
            pltpu.VMEM((1, _D * _K), jnp.float32),
        ],
        compiler_params=pltpu.CompilerParams(
            dimension_semantics=("arbitrary",)),
    )(features, dcol, cluster_probabilities, est_global, est_domains)
    return out_g, out_d


# speedup vs baseline: 90.7003x; 1.9357x over previous
import jax
import jax.numpy as jnp
from jax import lax
from jax.experimental import pallas as pl
from jax.experimental.pallas import tpu as pltpu

_B = 16384
_F = 128
_K = 32
_D = 4
_ALPHA = 0.9
_EPS = 0.001
_TILE = 2048
_NB = _B // _TILE


def _body(pt_ref, d_ref, f_ref, eg_ref, ed_ref, out_g_ref, out_d_ref,
          acc_ref):
    i = pl.program_id(0)

    @pl.when(i == 0)
    def _init():
        acc_ref[...] = jnp.zeros_like(acc_ref)

    pt = pt_ref[...]
    f = f_ref[...]
    drow = d_ref[0]
    f_aug = jnp.concatenate(
        [f, jnp.ones((_TILE, 8), jnp.float32)], axis=1)
    for d in range(_D):
        m = (drow == d).astype(jnp.float32)
        masked = pt * m
        acc_ref[d * _K:(d + 1) * _K, :] += lax.dot_general(
            masked, f_aug, (((1,), (0,)), ((), ())),
            preferred_element_type=jnp.float32)

    @pl.when(i == _NB - 1)
    def _finish():
        num_gt = jnp.zeros((_K, _F), jnp.float32)
        den_g = jnp.zeros((_K, 1), jnp.float32)
        for d in range(_D):
            num_dt = acc_ref[d * _K:(d + 1) * _K, 0:_F]
            den_d = acc_ref[d * _K:(d + 1) * _K, _F:_F + 1]
            num_gt += num_dt
            den_g += den_d
            cent_dt = num_dt / (den_d + _EPS)
            out_d_ref[d] = (ed_ref[d] * _ALPHA
                            + cent_dt.T * (1.0 - _ALPHA))
        cent_gt = num_gt / (den_g + _EPS)
        out_g_ref[...] = eg_ref[...] * _ALPHA + cent_gt.T * (1.0 - _ALPHA)


def kernel(features, domains, cluster_probabilities, est_global, est_domains):
    probs_t = cluster_probabilities.T
    dom3 = domains.reshape(_NB, 1, _TILE)
    out_g, out_d = pl.pallas_call(
        _body,
        grid=(_NB,),
        in_specs=[
            pl.BlockSpec((_K, _TILE), lambda i: (0, i)),
            pl.BlockSpec((1, 1, _TILE), lambda i: (i, 0, 0)),
            pl.BlockSpec((_TILE, _F), lambda i: (i, 0)),
            pl.BlockSpec((_F, _K), lambda i: (0, 0)),
            pl.BlockSpec((_D, _F, _K), lambda i: (0, 0, 0)),
        ],
        out_specs=[
            pl.BlockSpec((_F, _K), lambda i: (0, 0)),
            pl.BlockSpec((_D, _F, _K), lambda i: (0, 0, 0)),
        ],
        out_shape=[
            jax.ShapeDtypeStruct((_F, _K), jnp.float32),
            jax.ShapeDtypeStruct((_D, _F, _K), jnp.float32),
        ],
        scratch_shapes=[
            pltpu.VMEM((_D * _K, _F + 8), jnp.float32),
        ],
        compiler_params=pltpu.CompilerParams(
            dimension_semantics=("arbitrary",)),
    )(probs_t, dom3, features, est_global, est_domains)
    return out_g, out_d
